# baseline (speedup 1.0000x reference)
import functools

import jax
import jax.numpy as jnp
from jax import lax
from jax.experimental import pallas as pl
from jax.experimental.pallas import tpu as pltpu
from jax.experimental.pallas import tpu_sc as plsc

NE = 64
DM = 768
DF = 384
NT = 4096
TB = 128
MAXW = 96
NW = 32
TPW = NT // NW
RB = 1024
NRB = NT // RB
WT = 128


def _route_body(x_ref, r_ref, eid_ref, rank_ref, prob_ref,
                aoff_ref, we_ref, wxb_ref, runc_ref):
    i = pl.program_id(0)

    @pl.when(i == 0)
    def _():
        runc_ref[...] = jnp.zeros_like(runc_ref)

    xb = x_ref[0]
    logits = jnp.dot(xb, r_ref[...], preferred_element_type=jnp.float32)
    m = jnp.max(logits, axis=-1, keepdims=True)
    ssum = jnp.sum(jnp.exp(logits - m), axis=-1, keepdims=True)
    prob_ref[0] = 1.0 / ssum
    col = lax.broadcasted_iota(jnp.int32, (RB, NE), 1)
    eid = jnp.min(jnp.where(logits == m, col, NE), axis=-1)
    onehot = (col == eid[:, None]).astype(jnp.float32)
    r_iota = lax.broadcasted_iota(jnp.int32, (RB, RB), 0)
    c_iota = lax.broadcasted_iota(jnp.int32, (RB, RB), 1)
    tri = (r_iota >= c_iota).astype(jnp.float32)
    csum = jnp.dot(tri, onehot, preferred_element_type=jnp.float32)
    within = jnp.sum(csum * onehot, axis=-1) - 1.0
    runc = runc_ref[...]
    prev = jnp.sum(onehot * runc, axis=-1)
    eid_ref[0, 0] = eid
    rank_ref[0, 0] = (within + prev).astype(jnp.int32)
    runc_ref[...] = runc + jnp.sum(onehot, axis=0, keepdims=True)

    @pl.when(i == NRB - 1)
    def _():
        c = runc_ref[...]
        nb = jnp.ceil(c / TB)
        k_iota = lax.broadcasted_iota(jnp.int32, (NE, NE), 0)
        j_iota = lax.broadcasted_iota(jnp.int32, (NE, NE), 1)
        triu = (k_iota <= j_iota).astype(jnp.float32)
        bc = jnp.dot(nb, triu, preferred_element_type=jnp.float32)
        aoff_ref[...] = ((bc - nb) * TB).astype(jnp.int32)
        total = jnp.sum(nb)
        bcb = jnp.broadcast_to(bc, (WT, NE))
        irow = lax.broadcasted_iota(jnp.int32, (WT, NE), 0).astype(jnp.float32)
        wexp = jnp.sum((bcb <= irow).astype(jnp.float32), axis=-1)
        we_ref[...] = jnp.minimum(wexp, NE - 1).astype(jnp.int32)[:, None]
        irow1 = lax.broadcasted_iota(jnp.int32, (WT, 1), 0).astype(jnp.float32)
        wxb_ref[...] = jnp.minimum(irow1, total).astype(jnp.int32)


def _route(xb3, router):
    return pl.pallas_call(
        _route_body,
        grid=(NRB,),
        in_specs=[
            pl.BlockSpec((1, RB, DM), lambda i: (i, 0, 0)),
            pl.BlockSpec((DM, NE), lambda i: (0, 0)),
        ],
        out_specs=[
            pl.BlockSpec((1, 1, RB), lambda i: (i, 0, 0)),
            pl.BlockSpec((1, 1, RB), lambda i: (i, 0, 0)),
            pl.BlockSpec((1, RB, 1), lambda i: (i, 0, 0)),
            pl.BlockSpec((1, NE), lambda i: (0, 0)),
            pl.BlockSpec((WT, 1), lambda i: (0, 0)),
            pl.BlockSpec((WT, 1), lambda i: (0, 0)),
        ],
        out_shape=[
            jax.ShapeDtypeStruct((NRB, 1, RB), jnp.int32),
            jax.ShapeDtypeStruct((NRB, 1, RB), jnp.int32),
            jax.ShapeDtypeStruct((NRB, RB, 1), jnp.float32),
            jax.ShapeDtypeStruct((1, NE), jnp.int32),
            jax.ShapeDtypeStruct((WT, 1), jnp.int32),
            jax.ShapeDtypeStruct((WT, 1), jnp.int32),
        ],
        scratch_shapes=[pltpu.VMEM((1, NE), jnp.float32)],
        compiler_params=pltpu.CompilerParams(dimension_semantics=("arbitrary",)),
    )(xb3, router)


@functools.lru_cache(maxsize=None)
def _sc_mesh():
    return plsc.VectorSubcoreMesh(core_axis_name="c", subcore_axis_name="s",
                                  num_cores=2, num_subcores=16)


CH = 4
CR = TPW // CH


def _sc_scatter_body(eid_hbm, rank_hbm, aoff_hbm, x_hbm, xs_hbm, p_hbm,
                     eid_v, rank_v, off_v, p2_v, rows_v,
                     si0, si1, si2, si3, so0, so1, so2, so3):
    wid = lax.axis_index("s") * 2 + lax.axis_index("c")
    base = wid * TPW
    pltpu.sync_copy(eid_hbm.at[pl.ds(base, TPW)], eid_v)
    pltpu.sync_copy(rank_hbm.at[pl.ds(base, TPW)], rank_v)
    pltpu.sync_copy(aoff_hbm, off_v)
    sin = (si0, si1, si2, si3)
    sout = (so0, so1, so2, so3)
    incs = [pltpu.async_copy(x_hbm.at[pl.ds(base + j * CR, CR)],
                             rows_v.at[pl.ds(j * CR, CR)], sin[j])
            for j in range(CH)]
    for j in range(CH):
        for k in range(CR // 16):
            idx = eid_v[pl.ds(j * CR + k * 16, 16)]
            offs = plsc.load_gather(off_v, [idx])
            p2_v[j, pl.ds(k * 16, 16)] = (
                offs + rank_v[pl.ds(j * CR + k * 16, 16)])
    outcs = []
    for j in range(CH):
        incs[j].wait()
        outcs.append(pltpu.async_copy(rows_v.at[pl.ds(j * CR, CR)],
                                      xs_hbm.at[p2_v.at[j]], sout[j]))
    for c in outcs:
        c.wait()
    pltpu.sync_copy(p2_v, p_hbm.at[pl.ds(wid * CH, CH)])


@functools.lru_cache(maxsize=None)
def _sc_scatter():
    return pl.kernel(
        _sc_scatter_body,
        out_type=(
            jax.ShapeDtypeStruct((MAXW * TB, DM), jnp.float32),
            jax.ShapeDtypeStruct((NW * CH, CR), jnp.int32),
        ),
        mesh=_sc_mesh(),
        scratch_types=[
            pltpu.VMEM((TPW,), jnp.int32),
            pltpu.VMEM((TPW,), jnp.int32),
            pltpu.VMEM((NE,), jnp.int32),
            pltpu.VMEM((CH, CR), jnp.int32),
            pltpu.VMEM((TPW, DM), jnp.float32),
        ] + [pltpu.SemaphoreType.DMA] * (2 * CH),
        compiler_params=pltpu.CompilerParams(needs_layout_passes=False),
    )


def _ffn_body(we_ref, wxb_ref, x_ref, g_ref, u_ref, d_ref, o_ref):
    del we_ref
    i = pl.program_id(0)

    @pl.when(wxb_ref[i] == i)
    def _():
        xb = x_ref[...]
        g = jnp.dot(xb, g_ref[0], preferred_element_type=jnp.float32)
        u = jnp.dot(xb, u_ref[0], preferred_element_type=jnp.float32)
        h = g * jax.nn.sigmoid(g) * u
        o_ref[...] = jnp.dot(h, d_ref[0], preferred_element_type=jnp.float32)


def _ffn(we, wxb, xs, eg, eu, ed):
    grid_spec = pltpu.PrefetchScalarGridSpec(
        num_scalar_prefetch=2,
        grid=(MAXW,),
        in_specs=[
            pl.BlockSpec((TB, DM), lambda i, we, wxb: (wxb[i], 0)),
            pl.BlockSpec((1, DM, DF), lambda i, we, wxb: (we[i], 0, 0)),
            pl.BlockSpec((1, DM, DF), lambda i, we, wxb: (we[i], 0, 0)),
            pl.BlockSpec((1, DF, DM), lambda i, we, wxb: (we[i], 0, 0)),
        ],
        out_specs=pl.BlockSpec((TB, DM), lambda i, we, wxb: (wxb[i], 0)),
    )
    return pl.pallas_call(
        _ffn_body,
        grid_spec=grid_spec,
        out_shape=jax.ShapeDtypeStruct((MAXW * TB, DM), jnp.float32),
        compiler_params=pltpu.CompilerParams(dimension_semantics=("arbitrary",)),
    )(we, wxb, xs, eg, eu, ed)


def _sc_gather_body(p_hbm, os_hbm, out_hbm, p2_v, rows_v,
                    si0, si1, si2, si3, so0, so1, so2, so3):
    wid = lax.axis_index("s") * 2 + lax.axis_index("c")
    base = wid * TPW
    sin = (si0, si1, si2, si3)
    sout = (so0, so1, so2, so3)
    pltpu.sync_copy(p_hbm.at[pl.ds(wid * CH, CH)], p2_v)
    incs = [pltpu.async_copy(os_hbm.at[p2_v.at[j]],
                             rows_v.at[pl.ds(j * CR, CR)], sin[j])
            for j in range(CH)]
    outcs = []
    for j in range(CH):
        incs[j].wait()
        outcs.append(pltpu.async_copy(rows_v.at[pl.ds(j * CR, CR)],
                                      out_hbm.at[pl.ds(base + j * CR, CR)],
                                      sout[j]))
    for c in outcs:
        c.wait()


@functools.lru_cache(maxsize=None)
def _sc_gather():
    return pl.kernel(
        _sc_gather_body,
        out_type=jax.ShapeDtypeStruct((NT, DM), jnp.float32),
        mesh=_sc_mesh(),
        scratch_types=[
            pltpu.VMEM((CH, CR), jnp.int32),
            pltpu.VMEM((TPW, DM), jnp.float32),
        ] + [pltpu.SemaphoreType.DMA] * (2 * CH),
    )


def _combine_body(x_ref, rt_ref, pr_ref, sg_ref, su_ref, sd_ref, o_ref):
    xb = x_ref[0].astype(jnp.bfloat16)
    g = jnp.dot(xb, sg_ref[...], preferred_element_type=jnp.float32)
    u = jnp.dot(xb, su_ref[...], preferred_element_type=jnp.float32)
    sh = jnp.dot((g * jax.nn.sigmoid(g) * u).astype(jnp.bfloat16), sd_ref[...],
                 preferred_element_type=jnp.float32)
    o_ref[0] = sh + rt_ref[0] * pr_ref[0]


def _combine(xb3, rt3, prob3, sg, su, sd):
    return pl.pallas_call(
        _combine_body,
        grid=(NRB,),
        in_specs=[
            pl.BlockSpec((1, RB, DM), lambda i: (i, 0, 0)),
            pl.BlockSpec((1, RB, DM), lambda i: (i, 0, 0)),
            pl.BlockSpec((1, RB, 1), lambda i: (i, 0, 0)),
            pl.BlockSpec((DM, DF), lambda i: (0, 0)),
            pl.BlockSpec((DM, DF), lambda i: (0, 0)),
            pl.BlockSpec((DF, DM), lambda i: (0, 0)),
        ],
        out_specs=pl.BlockSpec((1, RB, DM), lambda i: (i, 0, 0)),
        out_shape=jax.ShapeDtypeStruct((NRB, RB, DM), jnp.float32),
        compiler_params=pltpu.CompilerParams(dimension_semantics=("arbitrary",)),
    )(xb3, rt3, prob3, sg, su, sd)


def kernel(x, router, experts_gate, experts_up, experts_down,
           shared_gate, shared_up, shared_down):
    batch, seq, _ = x.shape
    xf = x.reshape(NT, DM)
    xb3 = xf.reshape(NRB, RB, DM)
    eid3, rank3, prob3, aoff, we2, wxb2 = _route(xb3, router)
    we = we2.reshape(WT)[:MAXW]
    wxb = wxb2.reshape(WT)[:MAXW]
    xs, p = _sc_scatter()(eid3.reshape(NT), rank3.reshape(NT),
                          aoff.reshape(NE), xf)
    os_ = _ffn(we, wxb, xs, experts_gate, experts_up, experts_down)
    routed = _sc_gather()(p, os_)
    out = _combine(xb3, routed.reshape(NRB, RB, DM), prob3,
                   shared_gate.astype(jnp.bfloat16),
                   shared_up.astype(jnp.bfloat16),
                   shared_down.astype(jnp.bfloat16))
    return out.reshape(batch, seq, DM)


# speedup vs baseline: 1.0131x; 1.0131x over previous
import functools

import jax
import jax.numpy as jnp
from jax import lax
from jax.experimental import pallas as pl
from jax.experimental.pallas import tpu as pltpu
from jax.experimental.pallas import tpu_sc as plsc

NE = 64
DM = 768
DF = 384
NT = 4096
TB = 128
MAXW = 96
NW = 32
TPW = NT // NW
RB = 1024
NRB = NT // RB
WT = 128


def _route_body(x_ref, r_ref, eid_ref, rank_ref, prob_ref,
                aoff_ref, we_ref, wxb_ref, runc_ref):
    i = pl.program_id(0)

    @pl.when(i == 0)
    def _():
        runc_ref[...] = jnp.zeros_like(runc_ref)

    xb = x_ref[0]
    logits = jnp.dot(xb, r_ref[...], preferred_element_type=jnp.float32)
    m = jnp.max(logits, axis=-1, keepdims=True)
    ssum = jnp.sum(jnp.exp(logits - m), axis=-1, keepdims=True)
    prob_ref[0] = 1.0 / ssum
    col = lax.broadcasted_iota(jnp.int32, (RB, NE), 1)
    eid = jnp.min(jnp.where(logits == m, col, NE), axis=-1)
    onehot = (col == eid[:, None]).astype(jnp.float32)
    r_iota = lax.broadcasted_iota(jnp.int32, (RB, RB), 0)
    c_iota = lax.broadcasted_iota(jnp.int32, (RB, RB), 1)
    tri = (r_iota >= c_iota).astype(jnp.float32)
    csum = jnp.dot(tri, onehot, preferred_element_type=jnp.float32)
    within = jnp.sum(csum * onehot, axis=-1) - 1.0
    runc = runc_ref[...]
    prev = jnp.sum(onehot * runc, axis=-1)
    eid_ref[0, 0] = eid
    rank_ref[0, 0] = (within + prev).astype(jnp.int32)
    runc_ref[...] = runc + jnp.sum(onehot, axis=0, keepdims=True)

    @pl.when(i == NRB - 1)
    def _():
        c = runc_ref[...]
        nb = jnp.ceil(c / TB)
        k_iota = lax.broadcasted_iota(jnp.int32, (NE, NE), 0)
        j_iota = lax.broadcasted_iota(jnp.int32, (NE, NE), 1)
        triu = (k_iota <= j_iota).astype(jnp.float32)
        bc = jnp.dot(nb, triu, preferred_element_type=jnp.float32)
        aoff_ref[...] = ((bc - nb) * TB).astype(jnp.int32)
        total = jnp.sum(nb)
        bcb = jnp.broadcast_to(bc, (WT, NE))
        irow = lax.broadcasted_iota(jnp.int32, (WT, NE), 0).astype(jnp.float32)
        wexp = jnp.sum((bcb <= irow).astype(jnp.float32), axis=-1)
        we_ref[...] = jnp.minimum(wexp, NE - 1).astype(jnp.int32)[:, None]
        irow1 = lax.broadcasted_iota(jnp.int32, (WT, 1), 0).astype(jnp.float32)
        wxb_ref[...] = jnp.minimum(irow1, total).astype(jnp.int32)


def _route(xb3, router):
    return pl.pallas_call(
        _route_body,
        grid=(NRB,),
        in_specs=[
            pl.BlockSpec((1, RB, DM), lambda i: (i, 0, 0)),
            pl.BlockSpec((DM, NE), lambda i: (0, 0)),
        ],
        out_specs=[
            pl.BlockSpec((1, 1, RB), lambda i: (i, 0, 0)),
            pl.BlockSpec((1, 1, RB), lambda i: (i, 0, 0)),
            pl.BlockSpec((1, RB, 1), lambda i: (i, 0, 0)),
            pl.BlockSpec((1, NE), lambda i: (0, 0)),
            pl.BlockSpec((WT, 1), lambda i: (0, 0)),
            pl.BlockSpec((WT, 1), lambda i: (0, 0)),
        ],
        out_shape=[
            jax.ShapeDtypeStruct((NRB, 1, RB), jnp.int32),
            jax.ShapeDtypeStruct((NRB, 1, RB), jnp.int32),
            jax.ShapeDtypeStruct((NRB, RB, 1), jnp.float32),
            jax.ShapeDtypeStruct((1, NE), jnp.int32),
            jax.ShapeDtypeStruct((WT, 1), jnp.int32),
            jax.ShapeDtypeStruct((WT, 1), jnp.int32),
        ],
        scratch_shapes=[pltpu.VMEM((1, NE), jnp.float32)],
        compiler_params=pltpu.CompilerParams(dimension_semantics=("arbitrary",)),
    )(xb3, router)


@functools.lru_cache(maxsize=None)
def _sc_mesh():
    return plsc.VectorSubcoreMesh(core_axis_name="c", subcore_axis_name="s",
                                  num_cores=2, num_subcores=16)


def _sc_scatter_body(eid_hbm, rank_hbm, aoff_hbm, x_hbm, xs_hbm, p_hbm,
                     eid_v, rank_v, off_v, p_v, rows_v, sem):
    wid = lax.axis_index("s") * 2 + lax.axis_index("c")
    base = wid * TPW
    pltpu.sync_copy(eid_hbm.at[pl.ds(base, TPW)], eid_v)
    pltpu.sync_copy(rank_hbm.at[pl.ds(base, TPW)], rank_v)
    pltpu.sync_copy(aoff_hbm, off_v)
    for j in range(TPW // 16):
        idx = eid_v[pl.ds(j * 16, 16)]
        offs = plsc.load_gather(off_v, [idx])
        p_v[pl.ds(j * 16, 16)] = offs + rank_v[pl.ds(j * 16, 16)]
    pltpu.sync_copy(x_hbm.at[pl.ds(base, TPW)], rows_v)
    pltpu.async_copy(rows_v, xs_hbm.at[p_v], sem).wait()
    pltpu.sync_copy(p_v, p_hbm.at[pl.ds(base, TPW)])


@functools.lru_cache(maxsize=None)
def _sc_scatter():
    return pl.kernel(
        _sc_scatter_body,
        out_type=(
            jax.ShapeDtypeStruct((MAXW * TB, DM), jnp.float32),
            jax.ShapeDtypeStruct((NT,), jnp.int32),
        ),
        mesh=_sc_mesh(),
        scratch_types=[
            pltpu.VMEM((TPW,), jnp.int32),
            pltpu.VMEM((TPW,), jnp.int32),
            pltpu.VMEM((NE,), jnp.int32),
            pltpu.VMEM((TPW,), jnp.int32),
            pltpu.VMEM((TPW, DM), jnp.float32),
            pltpu.SemaphoreType.DMA,
        ],
        compiler_params=pltpu.CompilerParams(needs_layout_passes=False),
    )


def _ffn_body(we_ref, wxb_ref, x_ref, g_ref, u_ref, d_ref, o_ref):
    del we_ref
    i = pl.program_id(0)

    @pl.when(wxb_ref[i] == i)
    def _():
        xb = x_ref[...]
        g = jnp.dot(xb, g_ref[0], preferred_element_type=jnp.float32)
        u = jnp.dot(xb, u_ref[0], preferred_element_type=jnp.float32)
        h = g * jax.nn.sigmoid(g) * u
        o_ref[...] = jnp.dot(h, d_ref[0], preferred_element_type=jnp.float32)


def _ffn(we, wxb, xs, eg, eu, ed):
    grid_spec = pltpu.PrefetchScalarGridSpec(
        num_scalar_prefetch=2,
        grid=(MAXW,),
        in_specs=[
            pl.BlockSpec((TB, DM), lambda i, we, wxb: (wxb[i], 0)),
            pl.BlockSpec((1, DM, DF), lambda i, we, wxb: (we[i], 0, 0)),
            pl.BlockSpec((1, DM, DF), lambda i, we, wxb: (we[i], 0, 0)),
            pl.BlockSpec((1, DF, DM), lambda i, we, wxb: (we[i], 0, 0)),
        ],
        out_specs=pl.BlockSpec((TB, DM), lambda i, we, wxb: (wxb[i], 0)),
    )
    return pl.pallas_call(
        _ffn_body,
        grid_spec=grid_spec,
        out_shape=jax.ShapeDtypeStruct((MAXW * TB, DM), jnp.float32),
        compiler_params=pltpu.CompilerParams(dimension_semantics=("arbitrary",)),
    )(we, wxb, xs, eg, eu, ed)


def _sc_gather_body(p_hbm, os_hbm, out_hbm, p_v, rows_v, sem):
    wid = lax.axis_index("s") * 2 + lax.axis_index("c")
    base = wid * TPW
    pltpu.sync_copy(p_hbm.at[pl.ds(base, TPW)], p_v)
    pltpu.async_copy(os_hbm.at[p_v], rows_v, sem).wait()
    pltpu.sync_copy(rows_v, out_hbm.at[pl.ds(base, TPW)])


@functools.lru_cache(maxsize=None)
def _sc_gather():
    return pl.kernel(
        _sc_gather_body,
        out_type=jax.ShapeDtypeStruct((NT, DM), jnp.float32),
        mesh=_sc_mesh(),
        scratch_types=[
            pltpu.VMEM((TPW,), jnp.int32),
            pltpu.VMEM((TPW, DM), jnp.float32),
            pltpu.SemaphoreType.DMA,
        ],
    )


def _combine_body(x_ref, rt_ref, pr_ref, sg_ref, su_ref, sd_ref, o_ref):
    xb = x_ref[0].astype(jnp.bfloat16)
    g = jnp.dot(xb, sg_ref[...], preferred_element_type=jnp.float32)
    u = jnp.dot(xb, su_ref[...], preferred_element_type=jnp.float32)
    sh = jnp.dot((g * jax.nn.sigmoid(g) * u).astype(jnp.bfloat16), sd_ref[...],
                 preferred_element_type=jnp.float32)
    o_ref[0] = sh + rt_ref[0] * pr_ref[0]


def _combine(xb3, rt3, prob3, sg, su, sd):
    return pl.pallas_call(
        _combine_body,
        grid=(NRB,),
        in_specs=[
            pl.BlockSpec((1, RB, DM), lambda i: (i, 0, 0)),
            pl.BlockSpec((1, RB, DM), lambda i: (i, 0, 0)),
            pl.BlockSpec((1, RB, 1), lambda i: (i, 0, 0)),
            pl.BlockSpec((DM, DF), lambda i: (0, 0)),
            pl.BlockSpec((DM, DF), lambda i: (0, 0)),
            pl.BlockSpec((DF, DM), lambda i: (0, 0)),
        ],
        out_specs=pl.BlockSpec((1, RB, DM), lambda i: (i, 0, 0)),
        out_shape=jax.ShapeDtypeStruct((NRB, RB, DM), jnp.float32),
        compiler_params=pltpu.CompilerParams(dimension_semantics=("arbitrary",)),
    )(xb3, rt3, prob3, sg, su, sd)


def kernel(x, router, experts_gate, experts_up, experts_down,
           shared_gate, shared_up, shared_down):
    batch, seq, _ = x.shape
    xf = x.reshape(NT, DM)
    xb3 = xf.reshape(NRB, RB, DM)
    eid3, rank3, prob3, aoff, we2, wxb2 = _route(xb3, router)
    we = we2.reshape(WT)[:MAXW]
    wxb = wxb2.reshape(WT)[:MAXW]
    xs, p = _sc_scatter()(eid3.reshape(NT), rank3.reshape(NT),
                          aoff.reshape(NE), xf)
    os_ = _ffn(we, wxb, xs, experts_gate, experts_up, experts_down)
    routed = _sc_gather()(p, os_)
    out = _combine(xb3, routed.reshape(NRB, RB, DM), prob3,
                   shared_gate.astype(jnp.bfloat16),
                   shared_up.astype(jnp.bfloat16),
                   shared_down.astype(jnp.bfloat16))
    return out.reshape(batch, seq, DM)
